# baseline (speedup 1.0000x reference)
import jax
import jax.numpy as jnp
from jax import lax
from jax.experimental import pallas as pl
from jax.experimental.pallas import tpu as pltpu
from jax.experimental.pallas import tpu_sc as plsc

NUM_ROWS = 37
PAD_ROWS = 40
EMBED_DIM = 512
B_TOTAL = 4096 * 50

NC = 2
NS = 16
NW = NC * NS
B_PER_W = B_TOTAL // NW
CHUNK = 80
NCHUNKS = B_PER_W // CHUNK
LANES = 16
SLICES = EMBED_DIM // LANES


def _sc_gather(idx_chunked, table_pad):
    mesh = plsc.VectorSubcoreMesh(core_axis_name="c", subcore_axis_name="s")

    @pl.kernel(
        out_type=jax.ShapeDtypeStruct((B_TOTAL, EMBED_DIM), jnp.float32),
        mesh=mesh,
        scratch_types=[
            pltpu.VMEM((PAD_ROWS, EMBED_DIM), jnp.float32),
            pltpu.VMEM((NCHUNKS, CHUNK), jnp.int32),
            pltpu.VMEM((CHUNK, EMBED_DIM), jnp.float32),
            pltpu.VMEM((CHUNK, EMBED_DIM), jnp.float32),
            pltpu.SemaphoreType.DMA,
            pltpu.SemaphoreType.DMA,
        ],
    )
    def k(idx_hbm, table_hbm, out_hbm, table_v, idx_v, rows0, rows1, sem0, sem1):
        wid = lax.axis_index("s") * NC + lax.axis_index("c")
        base = wid * B_PER_W
        pltpu.sync_copy(table_hbm, table_v)
        pltpu.sync_copy(idx_hbm.at[wid], idx_v)

        col0 = jnp.arange(16, dtype=jnp.int32)

        zeros16 = col0 * 0

        def materialize(buf, chunk):
            def group(g, _):
                ivec = idx_v[chunk, pl.ds(g * LANES, LANES)]
                for j in range(LANES):
                    i = ivec[j]
                    for s in range(SLICES):
                        buf[g * LANES + j, pl.ds(LANES * s, LANES)] = (
                            table_v[i, pl.ds(LANES * s, LANES)])
                return _
            lax.fori_loop(0, CHUNK // LANES, group, None)

        def scatter_start(buf, chunk, sem):
            pltpu.async_copy(
                buf, out_hbm.at[pl.ds(base + chunk * CHUNK, CHUNK)], sem)

        def scatter_wait(buf, sem):
            pltpu.make_async_copy(
                buf, out_hbm.at[pl.ds(base, CHUNK)], sem).wait()

        materialize(rows0, 0)

        def pair(it, _):
            c = it * 2
            scatter_start(rows0, c, sem0)
            materialize(rows1, c + 1)
            scatter_wait(rows0, sem0)
            scatter_start(rows1, c + 1, sem1)
            materialize(rows0, c + 2)
            scatter_wait(rows1, sem1)
            return _

        lax.fori_loop(0, NCHUNKS // 2 - 1, pair, None)

        scatter_start(rows0, NCHUNKS - 2, sem0)
        materialize(rows1, NCHUNKS - 1)
        scatter_wait(rows0, sem0)
        scatter_start(rows1, NCHUNKS - 1, sem1)
        scatter_wait(rows1, sem1)

    return k(idx_chunked, table_pad)


def kernel(whitelist_tensor, table):
    idx_chunked = whitelist_tensor.astype(jnp.int32).reshape(NW, NCHUNKS, CHUNK)
    table_pad = jnp.pad(table, ((0, PAD_ROWS - NUM_ROWS), (0, 0)))
    out = _sc_gather(idx_chunked, table_pad)
    return out.reshape(whitelist_tensor.shape + (EMBED_DIM,))


# speedup vs baseline: 2.9128x; 2.9128x over previous
import jax
import jax.numpy as jnp
from jax import lax
from jax.experimental import pallas as pl

NUM_ROWS = 37
PAD_ROWS = 64
EMBED_DIM = 512
BATCH = 4096
SEQ = 50
B_BLK = 256


def _tc_gather(idx, table_pad):
    def body(idx_ref, tab_ref, out_ref):
        idxb = idx_ref[...]
        iota = lax.broadcasted_iota(jnp.int32, (B_BLK, SEQ, PAD_ROWS), 2)
        oh = (idxb[:, :, None] == iota).astype(jnp.float32)
        out_ref[...] = lax.dot_general(
            oh, tab_ref[...],
            dimension_numbers=(((2,), (0,)), ((), ())),
            preferred_element_type=jnp.float32)

    return pl.pallas_call(
        body,
        grid=(BATCH // B_BLK,),
        in_specs=[
            pl.BlockSpec((B_BLK, SEQ), lambda i: (i, 0)),
            pl.BlockSpec((PAD_ROWS, EMBED_DIM), lambda i: (0, 0)),
        ],
        out_specs=pl.BlockSpec((B_BLK, SEQ, EMBED_DIM), lambda i: (i, 0, 0)),
        out_shape=jax.ShapeDtypeStruct((BATCH, SEQ, EMBED_DIM), jnp.float32),
    )(idx, table_pad)


def kernel(whitelist_tensor, table):
    idx = whitelist_tensor.astype(jnp.int32)
    table_pad = jnp.pad(table, ((0, PAD_ROWS - NUM_ROWS), (0, 0)))
    return _tc_gather(idx, table_pad)


# speedup vs baseline: 2.9236x; 1.0037x over previous
import jax
import jax.numpy as jnp
from jax import lax
from jax.experimental import pallas as pl

NUM_ROWS = 37
PAD_ROWS = 64
EMBED_DIM = 512
BATCH = 4096
SEQ = 50
B_BLK = 128


def _tc_gather(idx, table_pad):
    def body(idx_ref, tab_ref, out_ref):
        idxb = idx_ref[...]
        iota = lax.broadcasted_iota(jnp.int32, (B_BLK, SEQ, PAD_ROWS), 2)
        oh = (idxb[:, :, None] == iota).astype(jnp.float32)
        out_ref[...] = lax.dot_general(
            oh, tab_ref[...],
            dimension_numbers=(((2,), (0,)), ((), ())),
            preferred_element_type=jnp.float32)

    return pl.pallas_call(
        body,
        grid=(BATCH // B_BLK,),
        in_specs=[
            pl.BlockSpec((B_BLK, SEQ), lambda i: (i, 0)),
            pl.BlockSpec((PAD_ROWS, EMBED_DIM), lambda i: (0, 0)),
        ],
        out_specs=pl.BlockSpec((B_BLK, SEQ, EMBED_DIM), lambda i: (i, 0, 0)),
        out_shape=jax.ShapeDtypeStruct((BATCH, SEQ, EMBED_DIM), jnp.float32),
    )(idx, table_pad)


def kernel(whitelist_tensor, table):
    idx = whitelist_tensor.astype(jnp.int32)
    table_pad = jnp.pad(table, ((0, PAD_ROWS - NUM_ROWS), (0, 0)))
    return _tc_gather(idx, table_pad)


# speedup vs baseline: 2.9237x; 1.0001x over previous
import jax
import jax.numpy as jnp
from jax import lax
from jax.experimental import pallas as pl
from jax.experimental.pallas import tpu as pltpu

NUM_ROWS = 37
PAD_ROWS = 64
EMBED_DIM = 512
BATCH = 4096
SEQ = 50
B_BLK = 128


def _tc_gather(idx, table_pad):
    def body(idx_ref, tab_ref, out_ref):
        idxb = idx_ref[...]
        iota = lax.broadcasted_iota(jnp.int32, (B_BLK, SEQ, PAD_ROWS), 2)
        oh = (idxb[:, :, None] == iota).astype(jnp.float32)
        out_ref[...] = lax.dot_general(
            oh, tab_ref[...],
            dimension_numbers=(((2,), (0,)), ((), ())),
            preferred_element_type=jnp.float32)

    return pl.pallas_call(
        body,
        grid=(BATCH // B_BLK,),
        in_specs=[
            pl.BlockSpec((B_BLK, SEQ), lambda i: (i, 0)),
            pl.BlockSpec((PAD_ROWS, EMBED_DIM), lambda i: (0, 0)),
        ],
        out_specs=pl.BlockSpec((B_BLK, SEQ, EMBED_DIM), lambda i: (i, 0, 0)),
        out_shape=jax.ShapeDtypeStruct((BATCH, SEQ, EMBED_DIM), jnp.float32),
        compiler_params=pltpu.CompilerParams(
            dimension_semantics=("parallel",)),
    )(idx, table_pad)


def kernel(whitelist_tensor, table):
    idx = whitelist_tensor.astype(jnp.int32)
    table_pad = jnp.pad(table, ((0, PAD_ROWS - NUM_ROWS), (0, 0)))
    return _tc_gather(idx, table_pad)
